# baseline (speedup 1.0000x reference)
import functools

import jax
import jax.numpy as jnp
from jax import lax
from jax.experimental import pallas as pl
from jax.experimental.pallas import tpu as pltpu
from jax.experimental.pallas import tpu_sc as plsc

_NC = 2
_NS = 16
_NW = _NC * _NS

_X = 4


def _emb_grid(batch, seq, d_model):
    seq_per_w = batch // _NW
    n_steps = seq_per_w // _X
    n_super = n_steps // 2
    mesh = plsc.VectorSubcoreMesh(core_axis_name="c", subcore_axis_name="s")

    @functools.partial(
        pl.kernel,
        mesh=mesh,
        out_type=jax.ShapeDtypeStruct((batch, seq, d_model), jnp.float32),
        scratch_types=[
            pltpu.VMEM((seq_per_w, seq), jnp.int32),
            pltpu.VMEM((2, _X, seq, d_model), jnp.float32),
            pltpu.SemaphoreType.DMA,
            pltpu.SemaphoreType.DMA,
            pltpu.SemaphoreType.DMA,
            pltpu.SemaphoreType.DMA,
        ],
        compiler_params=pltpu.CompilerParams(use_tc_tiling_on_sc=False),
    )
    def emb(idx_hbm, table_hbm, out_hbm, idx_v, rows_v, g0, g1, s0, s1):
        wid = lax.axis_index("s") * _NC + lax.axis_index("c")
        base_seq = wid * seq_per_w
        gsem = (g0, g1)
        ssem = (s0, s1)

        pltpu.sync_copy(idx_hbm.at[pl.ds(base_seq, seq_per_w)], idx_v)

        def fire_gathers(step, slot):
            for j in range(_X):
                pltpu.async_copy(
                    table_hbm.at[idx_v.at[step * _X + j]],
                    rows_v.at[slot, j],
                    gsem[slot],
                )

        def wait_gathers(slot):
            pltpu.make_async_copy(
                out_hbm.at[pl.ds(0, _X)], rows_v.at[slot], gsem[slot]
            ).wait()

        def fire_store(step, slot):
            pltpu.async_copy(
                rows_v.at[slot],
                out_hbm.at[pl.ds(base_seq + step * _X, _X)],
                ssem[slot],
            )

        def wait_store(slot):
            pltpu.make_async_copy(
                rows_v.at[slot], out_hbm.at[pl.ds(0, _X)], ssem[slot]
            ).wait()

        fire_gathers(0, 0)
        fire_gathers(1, 1)
        wait_gathers(0)
        fire_store(0, 0)

        def superstep(t, carry):
            for b in range(2):
                k = 2 * t + b
                wait_store(b)
                fire_gathers(k, b)
                wait_gathers(1 - b)
                fire_store(k - 1, 1 - b)
            return carry

        lax.fori_loop(1, n_super, superstep, 0)

        wait_gathers(1)
        fire_store(n_steps - 1, 1)
        wait_store(0)
        wait_store(1)

    return emb


def kernel(tokens, token_emb):
    b, s = tokens.shape
    v, d = token_emb.shape
    return _emb_grid(b, s, d)(tokens.astype(jnp.int32), token_emb)
